# baseline (speedup 1.0000x reference)
import functools

import jax
import jax.numpy as jnp
from jax import lax
from jax.experimental import pallas as pl
from jax.experimental.pallas import tpu as pltpu
from jax.experimental.pallas import tpu_sc as plsc

N = 20000
NP = 20480
PAD = NP - N
B = 256
NBLK = NP // B
T = 16
CHUNK = NP // T
VREGS = CHUNK // 16
BPAD = B + 16
SHMAX = (NP - B) // T
SPAD = NP + 1280
NMS_T = 0.3
SCORE_T = 0.05


def _iou_supp(ry1, rx1, ry2, rx2, ra, y1v, x1v, y2v, x2v, av):
    yy1 = jnp.maximum(ry1, y1v)
    xx1 = jnp.maximum(rx1, x1v)
    yy2 = jnp.minimum(ry2, y2v)
    xx2 = jnp.minimum(rx2, x2v)
    inter = jnp.maximum(yy2 - yy1, 0.0) * jnp.maximum(xx2 - xx1, 0.0)
    iou = inter / (ra + av - inter + 1e-9)
    return iou > NMS_T


def _make_kernel():
    mesh = plsc.VectorSubcoreMesh(
        core_axis_name="c", subcore_axis_name="s", num_cores=1, num_subcores=T
    )

    def body(y1h, x1h, y2h, x2h, sh, idxh,
             oy1, ox1, oy2, ox2, osv,
             sy1, sx1, sy2, sx2, sar, skp, ssv,
             by1, bx1, by2, bx2, bar, bkp,
             hy1, hx1, hy2, hx2, har, hkp,
             g1, g2, g3, g4, g5, ga, gk, gidx,
             my1, mx1, my2, mx2, mar, dsem, hsem):
        wid = lax.axis_index("s")
        base = pl.multiple_of(wid * CHUNK, 256)
        nrows = CHUNK // 128

        pltpu.sync_copy(idxh.at[wid], gidx)
        for c in range(nrows):
            dst = pl.ds(c * 128, 128)
            pltpu.sync_copy(y1h.at[gidx.at[c]], g1.at[dst])
            pltpu.sync_copy(x1h.at[gidx.at[c]], g2.at[dst])
            pltpu.sync_copy(y2h.at[gidx.at[c]], g3.at[dst])
            pltpu.sync_copy(x2h.at[gidx.at[c]], g4.at[dst])
            pltpu.sync_copy(sh.at[gidx.at[c]], g5.at[dst])

        def acomp(v, _):
            off = pl.ds(v * 16, 16)
            a = (jnp.maximum(g3[off] - g1[off], 0.0)
                 * jnp.maximum(g4[off] - g2[off], 0.0))
            ga[off] = a
            gk[off] = jnp.where(g5[off] > 0.0, 1.0, 0.0)
            return 0

        lax.fori_loop(0, VREGS, acomp, 0)

        dchunk = pl.ds(base, CHUNK)
        pltpu.sync_copy(g1, sy1.at[dchunk])
        pltpu.sync_copy(g2, sx1.at[dchunk])
        pltpu.sync_copy(g3, sy2.at[dchunk])
        pltpu.sync_copy(g4, sx2.at[dchunk])
        pltpu.sync_copy(ga, sar.at[dchunk])
        pltpu.sync_copy(gk, skp.at[dchunk])
        pltpu.sync_copy(g5, ssv.at[dchunk])
        plsc.subcore_barrier()

        def blk_body(blk, _):
            bs = pl.multiple_of(blk * B, 256)
            dblk = pl.ds(bs, B)
            head = pl.ds(0, B)
            s0 = bs + B
            share = (NP - s0) // T
            mystart = pl.multiple_of(s0 + wid * share, 16)
            nv = share // 16
            dsh = pl.ds(mystart, SHMAX)

            bdescs = [
                pltpu.async_copy(sy1.at[dblk], by1.at[head], dsem),
                pltpu.async_copy(sx1.at[dblk], bx1.at[head], dsem),
                pltpu.async_copy(sy2.at[dblk], by2.at[head], dsem),
                pltpu.async_copy(sx2.at[dblk], bx2.at[head], dsem),
                pltpu.async_copy(sar.at[dblk], bar.at[head], dsem),
                pltpu.async_copy(skp.at[dblk], bkp.at[head], dsem),
            ]
            sdescs = [
                pltpu.async_copy(sy1.at[dsh], hy1.at[pl.ds(0, SHMAX)], hsem),
                pltpu.async_copy(sx1.at[dsh], hx1.at[pl.ds(0, SHMAX)], hsem),
                pltpu.async_copy(sy2.at[dsh], hy2.at[pl.ds(0, SHMAX)], hsem),
                pltpu.async_copy(sx2.at[dsh], hx2.at[pl.ds(0, SHMAX)], hsem),
                pltpu.async_copy(sar.at[dsh], har.at[pl.ds(0, SHMAX)], hsem),
                pltpu.async_copy(skp.at[dsh], hkp.at[pl.ds(0, SHMAX)], hsem),
            ]
            for d in bdescs:
                d.wait()

            lane16 = lax.broadcasted_iota(jnp.int32, (16,), 0)

            def intra_r(r, nk):
                kr = bkp[pl.ds(r, 16)][0]

                def kept(nk):
                    ry1 = by1[pl.ds(r, 16)][0]
                    rx1 = bx1[pl.ds(r, 16)][0]
                    ry2 = by2[pl.ds(r, 16)][0]
                    rx2 = bx2[pl.ds(r, 16)][0]
                    ra = bar[pl.ds(r, 16)][0]
                    my1[nk] = ry1
                    mx1[nk] = rx1
                    my2[nk] = ry2
                    mx2[nk] = rx2
                    mar[nk] = ra

                    def iv(v2, _):
                        for h in range(2):
                            v = v2 * 2 + h
                            off = pl.ds(v * 16, 16)
                            supp = _iou_supp(ry1, rx1, ry2, rx2, ra,
                                             by1[off], bx1[off], by2[off],
                                             bx2[off], bar[off])
                            supp = supp & ((v * 16 + lane16) > r)
                            bkp[off] = jnp.where(supp, 0.0, bkp[off])
                        return 0

                    lax.fori_loop(r // 32, B // 32, iv, 0)
                    return nk + 1

                return lax.cond(kr > 0.0, kept, lambda nk: nk, nk)

            nk = lax.fori_loop(0, B, intra_r, 0)

            @pl.when(wid == lax.rem(blk, T))
            def _():
                pltpu.sync_copy(bkp.at[head], skp.at[dblk])

            nk4 = ((nk + 3) // 4) * 4

            def zpad(i, _):
                my1[i] = 0.0
                mx1[i] = 0.0
                my2[i] = 0.0
                mx2[i] = 0.0
                mar[i] = 0.0
                return 0

            lax.fori_loop(nk, nk4, zpad, 0)

            for d in sdescs:
                d.wait()

            def quad_body(k, _):
                rows = []
                for j in range(4):
                    i = 4 * k + j
                    rows.append((my1[i], mx1[i], my2[i], mx2[i], mar[i]))

                def sv_(v2, _):
                    for h in range(2):
                        off = pl.ds((v2 * 2 + h) * 16, 16)
                        y1v = hy1[off]; x1v = hx1[off]
                        y2v = hy2[off]; x2v = hx2[off]; av = har[off]
                        supp = None
                        for (ry1, rx1, ry2, rx2, ra) in rows:
                            s = _iou_supp(ry1, rx1, ry2, rx2, ra,
                                          y1v, x1v, y2v, x2v, av)
                            supp = s if supp is None else (supp | s)
                        hkp[off] = jnp.where(supp, 0.0, hkp[off])
                    return 0

                lax.fori_loop(0, (nv + 1) // 2, sv_, 0)
                return 0

            lax.fori_loop(0, nk4 // 4, quad_body, 0)

            for bit in (64, 32, 16, 8, 4, 2, 1):
                off_elems = (nv & ~(2 * bit - 1)) * 16

                @pl.when((nv & bit) != 0)
                def _(bit=bit, off_elems=off_elems):
                    pltpu.sync_copy(
                        hkp.at[pl.ds(pl.multiple_of(off_elems, 16), bit * 16)],
                        skp.at[pl.ds(pl.multiple_of(mystart + off_elems, 16),
                                     bit * 16)])

            plsc.subcore_barrier()
            return 0

        lax.fori_loop(0, NBLK, blk_body, 0)

        pltpu.sync_copy(sy1.at[dchunk], g1)
        pltpu.sync_copy(sx1.at[dchunk], g2)
        pltpu.sync_copy(sy2.at[dchunk], g3)
        pltpu.sync_copy(sx2.at[dchunk], g4)
        pltpu.sync_copy(ssv.at[dchunk], g5)
        pltpu.sync_copy(skp.at[dchunk], gk)

        def ccomp(v, _):
            off = pl.ds(v * 16, 16)
            m = gk[off] > 0.0
            g1[off] = jnp.where(m, g1[off], 0.0)
            g2[off] = jnp.where(m, g2[off], 0.0)
            g3[off] = jnp.where(m, g3[off], 0.0)
            g4[off] = jnp.where(m, g4[off], 0.0)
            g5[off] = jnp.where(m, g5[off], 0.0)
            return 0

        lax.fori_loop(0, VREGS, ccomp, 0)

        pltpu.sync_copy(g1, oy1.at[dchunk])
        pltpu.sync_copy(g2, ox1.at[dchunk])
        pltpu.sync_copy(g3, oy2.at[dchunk])
        pltpu.sync_copy(g4, ox2.at[dchunk])
        pltpu.sync_copy(g5, osv.at[dchunk])

    f32 = jnp.float32
    out_type = [jax.ShapeDtypeStruct((NP,), f32) for _ in range(5)]
    scratch = (
        [pltpu.VMEM_SHARED((SPAD,), f32) for _ in range(7)]
        + [pltpu.VMEM((BPAD,), f32) for _ in range(6)]
        + [pltpu.VMEM((SHMAX + 16,), f32) for _ in range(6)]
        + [pltpu.VMEM((CHUNK,), f32) for _ in range(7)]
        + [pltpu.VMEM((CHUNK // 128, 128), jnp.int32)]
        + [pltpu.SMEM((B + 4,), f32) for _ in range(5)]
        + [pltpu.SemaphoreType.DMA, pltpu.SemaphoreType.DMA]
    )
    return pl.kernel(body, out_type=out_type, mesh=mesh, scratch_types=scratch)


_nms_sc = _make_kernel()


@jax.jit
def kernel(boxes, scores):
    s = jnp.where(scores > SCORE_T, scores, -1.0)
    s_pad = jnp.concatenate([s, jnp.full((PAD,), -1.0, jnp.float32)])
    order = jnp.argsort(-s_pad).astype(jnp.int32)
    idxh = order.reshape(T, CHUNK // 128, 128)
    bp = jnp.concatenate([boxes, jnp.zeros((PAD, 4), jnp.float32)], axis=0)
    y1, x1, y2, x2, sv = _nms_sc(
        bp[:, 0], bp[:, 1], bp[:, 2], bp[:, 3], s_pad, idxh)
    return jnp.stack([y1, x1, y2, x2, sv], axis=1)[:N]


# speedup vs baseline: 1.0273x; 1.0273x over previous
import functools

import jax
import jax.numpy as jnp
from jax import lax
from jax.experimental import pallas as pl
from jax.experimental.pallas import tpu as pltpu
from jax.experimental.pallas import tpu_sc as plsc

N = 20000
NP = 20480
PAD = NP - N
B = 256
NBLK = NP // B
T = 16
CHUNK = NP // T
VREGS = CHUNK // 16
BPAD = B + 16
SHMAX = (NP - B) // T
SPAD = NP + 1280
NMS_T = 0.3
SCORE_T = 0.05


def _iou_supp(ry1, rx1, ry2, rx2, ra, y1v, x1v, y2v, x2v, av):
    yy1 = jnp.maximum(ry1, y1v)
    xx1 = jnp.maximum(rx1, x1v)
    yy2 = jnp.minimum(ry2, y2v)
    xx2 = jnp.minimum(rx2, x2v)
    inter = jnp.maximum(yy2 - yy1, 0.0) * jnp.maximum(xx2 - xx1, 0.0)
    iou = inter / (ra + av - inter + 1e-9)
    return iou > NMS_T


def _make_kernel():
    mesh = plsc.VectorSubcoreMesh(
        core_axis_name="c", subcore_axis_name="s", num_cores=1, num_subcores=T
    )

    def body(y1h, x1h, y2h, x2h, sh, idxh,
             oy1, ox1, oy2, ox2, osv,
             sy1, sx1, sy2, sx2, sar, skp, ssv,
             by1, bx1, by2, bx2, bar, bkp,
             hy1, hx1, hy2, hx2, har, hkp,
             g1, g2, g3, g4, g5, ga, gk, gidx,
             my1, mx1, my2, mx2, mar, dsem, hsem):
        wid = lax.axis_index("s")
        base = pl.multiple_of(wid * CHUNK, 256)
        nrows = CHUNK // 128

        pltpu.sync_copy(idxh.at[wid], gidx)
        for c in range(nrows):
            dst = pl.ds(c * 128, 128)
            pltpu.sync_copy(y1h.at[gidx.at[c]], g1.at[dst])
            pltpu.sync_copy(x1h.at[gidx.at[c]], g2.at[dst])
            pltpu.sync_copy(y2h.at[gidx.at[c]], g3.at[dst])
            pltpu.sync_copy(x2h.at[gidx.at[c]], g4.at[dst])
            pltpu.sync_copy(sh.at[gidx.at[c]], g5.at[dst])

        def acomp(v, _):
            off = pl.ds(v * 16, 16)
            a = (jnp.maximum(g3[off] - g1[off], 0.0)
                 * jnp.maximum(g4[off] - g2[off], 0.0))
            ga[off] = a
            gk[off] = jnp.where(g5[off] > 0.0, 1.0, 0.0)
            return 0

        lax.fori_loop(0, VREGS, acomp, 0)

        dchunk = pl.ds(base, CHUNK)
        pltpu.sync_copy(g1, sy1.at[dchunk])
        pltpu.sync_copy(g2, sx1.at[dchunk])
        pltpu.sync_copy(g3, sy2.at[dchunk])
        pltpu.sync_copy(g4, sx2.at[dchunk])
        pltpu.sync_copy(ga, sar.at[dchunk])
        pltpu.sync_copy(gk, skp.at[dchunk])
        pltpu.sync_copy(g5, ssv.at[dchunk])
        plsc.subcore_barrier()

        def blk_body(blk, _):
            bs = pl.multiple_of(blk * B, 256)
            dblk = pl.ds(bs, B)
            head = pl.ds(0, B)
            s0 = bs + B
            share = (NP - s0) // T
            mystart = pl.multiple_of(s0 + wid * share, 16)
            nv = share // 16
            dsh = pl.ds(mystart, SHMAX)

            bdescs = [
                pltpu.async_copy(sy1.at[dblk], by1.at[head], dsem),
                pltpu.async_copy(sx1.at[dblk], bx1.at[head], dsem),
                pltpu.async_copy(sy2.at[dblk], by2.at[head], dsem),
                pltpu.async_copy(sx2.at[dblk], bx2.at[head], dsem),
                pltpu.async_copy(sar.at[dblk], bar.at[head], dsem),
                pltpu.async_copy(skp.at[dblk], bkp.at[head], dsem),
            ]
            sdescs = [
                pltpu.async_copy(sy1.at[dsh], hy1.at[pl.ds(0, SHMAX)], hsem),
                pltpu.async_copy(sx1.at[dsh], hx1.at[pl.ds(0, SHMAX)], hsem),
                pltpu.async_copy(sy2.at[dsh], hy2.at[pl.ds(0, SHMAX)], hsem),
                pltpu.async_copy(sx2.at[dsh], hx2.at[pl.ds(0, SHMAX)], hsem),
                pltpu.async_copy(sar.at[dsh], har.at[pl.ds(0, SHMAX)], hsem),
                pltpu.async_copy(skp.at[dsh], hkp.at[pl.ds(0, SHMAX)], hsem),
            ]
            for d in bdescs:
                d.wait()

            lane16 = lax.broadcasted_iota(jnp.int32, (16,), 0)

            def intra_r(r, nk):
                kr = bkp[pl.ds(r, 16)][0]

                def kept(nk):
                    ry1 = by1[pl.ds(r, 16)][0]
                    rx1 = bx1[pl.ds(r, 16)][0]
                    ry2 = by2[pl.ds(r, 16)][0]
                    rx2 = bx2[pl.ds(r, 16)][0]
                    ra = bar[pl.ds(r, 16)][0]
                    my1[nk] = ry1
                    mx1[nk] = rx1
                    my2[nk] = ry2
                    mx2[nk] = rx2
                    mar[nk] = ra

                    def iv(v, _):
                        off = pl.ds(v * 16, 16)
                        supp = _iou_supp(ry1, rx1, ry2, rx2, ra,
                                         by1[off], bx1[off], by2[off],
                                         bx2[off], bar[off])
                        supp = supp & ((v * 16 + lane16) > r)
                        bkp[off] = jnp.where(supp, 0.0, bkp[off])
                        return 0

                    lax.fori_loop(r // 16, B // 16, iv, 0)
                    return nk + 1

                return lax.cond(kr > 0.0, kept, lambda nk: nk, nk)

            nk = lax.fori_loop(0, B, intra_r, 0)

            @pl.when(wid == lax.rem(blk, T))
            def _():
                pltpu.sync_copy(bkp.at[head], skp.at[dblk])

            nk4 = ((nk + 3) // 4) * 4

            def zpad(i, _):
                my1[i] = 0.0
                mx1[i] = 0.0
                my2[i] = 0.0
                mx2[i] = 0.0
                mar[i] = 0.0
                return 0

            lax.fori_loop(nk, nk4, zpad, 0)

            for d in sdescs:
                d.wait()

            def quad_body(k, _):
                rows = []
                for j in range(4):
                    i = 4 * k + j
                    rows.append((my1[i], mx1[i], my2[i], mx2[i], mar[i]))

                def sv_(v2, _):
                    for h in range(2):
                        off = pl.ds((v2 * 2 + h) * 16, 16)
                        y1v = hy1[off]; x1v = hx1[off]
                        y2v = hy2[off]; x2v = hx2[off]; av = har[off]
                        supp = None
                        for (ry1, rx1, ry2, rx2, ra) in rows:
                            s = _iou_supp(ry1, rx1, ry2, rx2, ra,
                                          y1v, x1v, y2v, x2v, av)
                            supp = s if supp is None else (supp | s)
                        hkp[off] = jnp.where(supp, 0.0, hkp[off])
                    return 0

                lax.fori_loop(0, (nv + 1) // 2, sv_, 0)
                return 0

            lax.fori_loop(0, nk4 // 4, quad_body, 0)

            for bit in (64, 32, 16, 8, 4, 2, 1):
                off_elems = (nv & ~(2 * bit - 1)) * 16

                @pl.when((nv & bit) != 0)
                def _(bit=bit, off_elems=off_elems):
                    pltpu.sync_copy(
                        hkp.at[pl.ds(pl.multiple_of(off_elems, 16), bit * 16)],
                        skp.at[pl.ds(pl.multiple_of(mystart + off_elems, 16),
                                     bit * 16)])

            plsc.subcore_barrier()
            return 0

        lax.fori_loop(0, NBLK, blk_body, 0)

        pltpu.sync_copy(sy1.at[dchunk], g1)
        pltpu.sync_copy(sx1.at[dchunk], g2)
        pltpu.sync_copy(sy2.at[dchunk], g3)
        pltpu.sync_copy(sx2.at[dchunk], g4)
        pltpu.sync_copy(ssv.at[dchunk], g5)
        pltpu.sync_copy(skp.at[dchunk], gk)

        def ccomp(v, _):
            off = pl.ds(v * 16, 16)
            m = gk[off] > 0.0
            g1[off] = jnp.where(m, g1[off], 0.0)
            g2[off] = jnp.where(m, g2[off], 0.0)
            g3[off] = jnp.where(m, g3[off], 0.0)
            g4[off] = jnp.where(m, g4[off], 0.0)
            g5[off] = jnp.where(m, g5[off], 0.0)
            return 0

        lax.fori_loop(0, VREGS, ccomp, 0)

        pltpu.sync_copy(g1, oy1.at[dchunk])
        pltpu.sync_copy(g2, ox1.at[dchunk])
        pltpu.sync_copy(g3, oy2.at[dchunk])
        pltpu.sync_copy(g4, ox2.at[dchunk])
        pltpu.sync_copy(g5, osv.at[dchunk])

    f32 = jnp.float32
    out_type = [jax.ShapeDtypeStruct((NP,), f32) for _ in range(5)]
    scratch = (
        [pltpu.VMEM_SHARED((SPAD,), f32) for _ in range(7)]
        + [pltpu.VMEM((BPAD,), f32) for _ in range(6)]
        + [pltpu.VMEM((SHMAX + 16,), f32) for _ in range(6)]
        + [pltpu.VMEM((CHUNK,), f32) for _ in range(7)]
        + [pltpu.VMEM((CHUNK // 128, 128), jnp.int32)]
        + [pltpu.SMEM((B + 4,), f32) for _ in range(5)]
        + [pltpu.SemaphoreType.DMA, pltpu.SemaphoreType.DMA]
    )
    return pl.kernel(body, out_type=out_type, mesh=mesh, scratch_types=scratch)


_nms_sc = _make_kernel()


@jax.jit
def kernel(boxes, scores):
    s = jnp.where(scores > SCORE_T, scores, -1.0)
    s_pad = jnp.concatenate([s, jnp.full((PAD,), -1.0, jnp.float32)])
    order = jnp.argsort(-s_pad).astype(jnp.int32)
    idxh = order.reshape(T, CHUNK // 128, 128)
    bp = jnp.concatenate([boxes, jnp.zeros((PAD, 4), jnp.float32)], axis=0)
    y1, x1, y2, x2, sv = _nms_sc(
        bp[:, 0], bp[:, 1], bp[:, 2], bp[:, 3], s_pad, idxh)
    return jnp.stack([y1, x1, y2, x2, sv], axis=1)[:N]


# speedup vs baseline: 1.1395x; 1.1093x over previous
import functools

import jax
import jax.numpy as jnp
from jax import lax
from jax.experimental import pallas as pl
from jax.experimental.pallas import tpu as pltpu
from jax.experimental.pallas import tpu_sc as plsc

N = 20000
NP = 20480
PAD = NP - N
B = 256
NBLK = NP // B
T = 16
CHUNK = NP // T
VREGS = CHUNK // 16
BPAD = B + 16
SHMAX = (NP - B) // T
SPAD = NP + 1280
NMS_T = 0.3
SCORE_T = 0.05


def _iou_supp(ry1, rx1, ry2, rx2, ra, y1v, x1v, y2v, x2v, av):
    yy1 = jnp.maximum(ry1, y1v)
    xx1 = jnp.maximum(rx1, x1v)
    yy2 = jnp.minimum(ry2, y2v)
    xx2 = jnp.minimum(rx2, x2v)
    inter = jnp.maximum(yy2 - yy1, 0.0) * jnp.maximum(xx2 - xx1, 0.0)
    iou = inter / (ra + av - inter + 1e-9)
    return iou > NMS_T


def _make_kernel():
    mesh = plsc.VectorSubcoreMesh(
        core_axis_name="c", subcore_axis_name="s", num_cores=1, num_subcores=T
    )

    def body(y1h, x1h, y2h, x2h, sh, idxh,
             oy1, ox1, oy2, ox2, osv,
             sy1, sx1, sy2, sx2, sar, skp, ssv,
             by1, bx1, by2, bx2, bar, bkp,
             hy1, hx1, hy2, hx2, har, hkp,
             g1, g2, g3, g4, g5, ga, gk, gidx,
             my1, mx1, my2, mx2, mar, dsem, hsem):
        wid = lax.axis_index("s")
        base = pl.multiple_of(wid * CHUNK, 256)
        nrows = CHUNK // 128

        pltpu.sync_copy(idxh.at[wid], gidx)
        for c in range(nrows):
            dst = pl.ds(c * 128, 128)
            pltpu.sync_copy(y1h.at[gidx.at[c]], g1.at[dst])
            pltpu.sync_copy(x1h.at[gidx.at[c]], g2.at[dst])
            pltpu.sync_copy(y2h.at[gidx.at[c]], g3.at[dst])
            pltpu.sync_copy(x2h.at[gidx.at[c]], g4.at[dst])
            pltpu.sync_copy(sh.at[gidx.at[c]], g5.at[dst])

        def acomp(v, _):
            off = pl.ds(v * 16, 16)
            a = (jnp.maximum(g3[off] - g1[off], 0.0)
                 * jnp.maximum(g4[off] - g2[off], 0.0))
            ga[off] = a
            gk[off] = jnp.where(g5[off] > 0.0, 1.0, 0.0)
            return 0

        lax.fori_loop(0, VREGS, acomp, 0)

        dchunk = pl.ds(base, CHUNK)
        pltpu.sync_copy(g1, sy1.at[dchunk])
        pltpu.sync_copy(g2, sx1.at[dchunk])
        pltpu.sync_copy(g3, sy2.at[dchunk])
        pltpu.sync_copy(g4, sx2.at[dchunk])
        pltpu.sync_copy(ga, sar.at[dchunk])
        pltpu.sync_copy(gk, skp.at[dchunk])
        pltpu.sync_copy(g5, ssv.at[dchunk])
        plsc.subcore_barrier()

        def blk_body(blk, _):
            bs = pl.multiple_of(blk * B, 256)
            dblk = pl.ds(bs, B)
            head = pl.ds(0, B)
            s0 = bs + B
            share = (NP - s0) // T
            mystart = pl.multiple_of(s0 + wid * share, 16)
            nv = share // 16
            dsh = pl.ds(mystart, SHMAX)

            bdescs = [
                pltpu.async_copy(sy1.at[dblk], by1.at[head], dsem),
                pltpu.async_copy(sx1.at[dblk], bx1.at[head], dsem),
                pltpu.async_copy(sy2.at[dblk], by2.at[head], dsem),
                pltpu.async_copy(sx2.at[dblk], bx2.at[head], dsem),
                pltpu.async_copy(sar.at[dblk], bar.at[head], dsem),
                pltpu.async_copy(skp.at[dblk], bkp.at[head], dsem),
            ]
            sdescs = [
                pltpu.async_copy(sy1.at[dsh], hy1.at[pl.ds(0, SHMAX)], hsem),
                pltpu.async_copy(sx1.at[dsh], hx1.at[pl.ds(0, SHMAX)], hsem),
                pltpu.async_copy(sy2.at[dsh], hy2.at[pl.ds(0, SHMAX)], hsem),
                pltpu.async_copy(sx2.at[dsh], hx2.at[pl.ds(0, SHMAX)], hsem),
                pltpu.async_copy(sar.at[dsh], har.at[pl.ds(0, SHMAX)], hsem),
                pltpu.async_copy(skp.at[dsh], hkp.at[pl.ds(0, SHMAX)], hsem),
            ]
            for d in bdescs:
                d.wait()

            lane16 = lax.broadcasted_iota(jnp.int32, (16,), 0)

            def intra_r(r, nk):
                kr = bkp[pl.ds(r, 16)][0]

                def kept(nk):
                    ry1 = by1[pl.ds(r, 16)][0]
                    rx1 = bx1[pl.ds(r, 16)][0]
                    ry2 = by2[pl.ds(r, 16)][0]
                    rx2 = bx2[pl.ds(r, 16)][0]
                    ra = bar[pl.ds(r, 16)][0]
                    my1[nk] = ry1
                    mx1[nk] = rx1
                    my2[nk] = ry2
                    mx2[nk] = rx2
                    mar[nk] = ra

                    def iv(v, _):
                        off = pl.ds(v * 16, 16)
                        supp = _iou_supp(ry1, rx1, ry2, rx2, ra,
                                         by1[off], bx1[off], by2[off],
                                         bx2[off], bar[off])
                        supp = supp & ((v * 16 + lane16) > r)
                        bkp[off] = jnp.where(supp, 0.0, bkp[off])
                        return 0

                    lax.fori_loop(r // 16, B // 16, iv, 0)
                    return nk + 1

                return lax.cond(kr > 0.0, kept, lambda nk: nk, nk)

            nk = lax.fori_loop(0, B, intra_r, 0)

            @pl.when(wid == lax.rem(blk, T))
            def _():
                pltpu.sync_copy(bkp.at[head], skp.at[dblk])

            nk4 = ((nk + 5) // 6) * 6

            def zpad(i, _):
                my1[i] = 0.0
                mx1[i] = 0.0
                my2[i] = 0.0
                mx2[i] = 0.0
                mar[i] = 0.0
                return 0

            lax.fori_loop(nk, nk4, zpad, 0)

            for d in sdescs:
                d.wait()

            def quad_body(k, _):
                rows = []
                for j in range(6):
                    i = 6 * k + j
                    rows.append((my1[i], mx1[i], my2[i], mx2[i], mar[i]))

                def sv_(v2, _):
                    for h in range(2):
                        off = pl.ds((v2 * 2 + h) * 16, 16)
                        y1v = hy1[off]; x1v = hx1[off]
                        y2v = hy2[off]; x2v = hx2[off]; av = har[off]
                        supp = None
                        for (ry1, rx1, ry2, rx2, ra) in rows:
                            s = _iou_supp(ry1, rx1, ry2, rx2, ra,
                                          y1v, x1v, y2v, x2v, av)
                            supp = s if supp is None else (supp | s)
                        hkp[off] = jnp.where(supp, 0.0, hkp[off])
                    return 0

                lax.fori_loop(0, (nv + 1) // 2, sv_, 0)
                return 0

            lax.fori_loop(0, nk4 // 6, quad_body, 0)

            for bit in (64, 32, 16, 8, 4, 2, 1):
                off_elems = (nv & ~(2 * bit - 1)) * 16

                @pl.when((nv & bit) != 0)
                def _(bit=bit, off_elems=off_elems):
                    pltpu.sync_copy(
                        hkp.at[pl.ds(pl.multiple_of(off_elems, 16), bit * 16)],
                        skp.at[pl.ds(pl.multiple_of(mystart + off_elems, 16),
                                     bit * 16)])

            plsc.subcore_barrier()
            return 0

        lax.fori_loop(0, NBLK, blk_body, 0)

        pltpu.sync_copy(sy1.at[dchunk], g1)
        pltpu.sync_copy(sx1.at[dchunk], g2)
        pltpu.sync_copy(sy2.at[dchunk], g3)
        pltpu.sync_copy(sx2.at[dchunk], g4)
        pltpu.sync_copy(ssv.at[dchunk], g5)
        pltpu.sync_copy(skp.at[dchunk], gk)

        def ccomp(v, _):
            off = pl.ds(v * 16, 16)
            m = gk[off] > 0.0
            g1[off] = jnp.where(m, g1[off], 0.0)
            g2[off] = jnp.where(m, g2[off], 0.0)
            g3[off] = jnp.where(m, g3[off], 0.0)
            g4[off] = jnp.where(m, g4[off], 0.0)
            g5[off] = jnp.where(m, g5[off], 0.0)
            return 0

        lax.fori_loop(0, VREGS, ccomp, 0)

        pltpu.sync_copy(g1, oy1.at[dchunk])
        pltpu.sync_copy(g2, ox1.at[dchunk])
        pltpu.sync_copy(g3, oy2.at[dchunk])
        pltpu.sync_copy(g4, ox2.at[dchunk])
        pltpu.sync_copy(g5, osv.at[dchunk])

    f32 = jnp.float32
    out_type = [jax.ShapeDtypeStruct((NP,), f32) for _ in range(5)]
    scratch = (
        [pltpu.VMEM_SHARED((SPAD,), f32) for _ in range(7)]
        + [pltpu.VMEM((BPAD,), f32) for _ in range(6)]
        + [pltpu.VMEM((SHMAX + 16,), f32) for _ in range(6)]
        + [pltpu.VMEM((CHUNK,), f32) for _ in range(7)]
        + [pltpu.VMEM((CHUNK // 128, 128), jnp.int32)]
        + [pltpu.SMEM((B + 6,), f32) for _ in range(5)]
        + [pltpu.SemaphoreType.DMA, pltpu.SemaphoreType.DMA]
    )
    return pl.kernel(body, out_type=out_type, mesh=mesh, scratch_types=scratch)


_nms_sc = _make_kernel()


@jax.jit
def kernel(boxes, scores):
    s = jnp.where(scores > SCORE_T, scores, -1.0)
    s_pad = jnp.concatenate([s, jnp.full((PAD,), -1.0, jnp.float32)])
    order = jnp.argsort(-s_pad).astype(jnp.int32)
    idxh = order.reshape(T, CHUNK // 128, 128)
    bp = jnp.concatenate([boxes, jnp.zeros((PAD, 4), jnp.float32)], axis=0)
    y1, x1, y2, x2, sv = _nms_sc(
        bp[:, 0], bp[:, 1], bp[:, 2], bp[:, 3], s_pad, idxh)
    return jnp.stack([y1, x1, y2, x2, sv], axis=1)[:N]


# speedup vs baseline: 1.1918x; 1.0459x over previous
import functools

import jax
import jax.numpy as jnp
from jax import lax
from jax.experimental import pallas as pl
from jax.experimental.pallas import tpu as pltpu
from jax.experimental.pallas import tpu_sc as plsc

N = 20000
NP = 20480
PAD = NP - N
B = 256
NBLK = NP // B
T = 16
CHUNK = NP // T
VREGS = CHUNK // 16
BPAD = B + 16
SHMAX = (NP - B) // T
SPAD = NP + 1280
NMS_T = 0.3
SCORE_T = 0.05


def _iou_supp(ry1, rx1, ry2, rx2, ra, y1v, x1v, y2v, x2v, av):
    yy1 = jnp.maximum(ry1, y1v)
    xx1 = jnp.maximum(rx1, x1v)
    yy2 = jnp.minimum(ry2, y2v)
    xx2 = jnp.minimum(rx2, x2v)
    inter = jnp.maximum(yy2 - yy1, 0.0) * jnp.maximum(xx2 - xx1, 0.0)
    iou = inter / (ra + av - inter + 1e-9)
    return iou > NMS_T


def _make_kernel():
    mesh = plsc.VectorSubcoreMesh(
        core_axis_name="c", subcore_axis_name="s", num_cores=1, num_subcores=T
    )

    def body(y1h, x1h, y2h, x2h, sh, idxh,
             oy1, ox1, oy2, ox2, osv,
             sy1, sx1, sy2, sx2, sar, skp, ssv,
             by1, bx1, by2, bx2, bar, bkp,
             hy1, hx1, hy2, hx2, har, hkp,
             g1, g2, g3, g4, g5, ga, gk, gidx,
             my1, mx1, my2, mx2, mar, dsem, hsem):
        wid = lax.axis_index("s")
        base = pl.multiple_of(wid * CHUNK, 256)
        nrows = CHUNK // 128

        pltpu.sync_copy(idxh.at[wid], gidx)
        for c in range(nrows):
            dst = pl.ds(c * 128, 128)
            pltpu.sync_copy(y1h.at[gidx.at[c]], g1.at[dst])
            pltpu.sync_copy(x1h.at[gidx.at[c]], g2.at[dst])
            pltpu.sync_copy(y2h.at[gidx.at[c]], g3.at[dst])
            pltpu.sync_copy(x2h.at[gidx.at[c]], g4.at[dst])
            pltpu.sync_copy(sh.at[gidx.at[c]], g5.at[dst])

        def acomp(v, _):
            off = pl.ds(v * 16, 16)
            a = (jnp.maximum(g3[off] - g1[off], 0.0)
                 * jnp.maximum(g4[off] - g2[off], 0.0))
            ga[off] = a
            gk[off] = jnp.where(g5[off] > 0.0, 1.0, 0.0)
            return 0

        lax.fori_loop(0, VREGS, acomp, 0)

        dchunk = pl.ds(base, CHUNK)
        pltpu.sync_copy(g1, sy1.at[dchunk])
        pltpu.sync_copy(g2, sx1.at[dchunk])
        pltpu.sync_copy(g3, sy2.at[dchunk])
        pltpu.sync_copy(g4, sx2.at[dchunk])
        pltpu.sync_copy(ga, sar.at[dchunk])
        pltpu.sync_copy(gk, skp.at[dchunk])
        pltpu.sync_copy(g5, ssv.at[dchunk])
        plsc.subcore_barrier()

        def blk_body(blk, _):
            bs = pl.multiple_of(blk * B, 256)
            dblk = pl.ds(bs, B)
            head = pl.ds(0, B)
            s0 = bs + B
            share = (NP - s0) // T
            mystart = pl.multiple_of(s0 + wid * share, 16)
            nv = share // 16
            dsh = pl.ds(mystart, SHMAX)

            bdescs = [
                pltpu.async_copy(sy1.at[dblk], by1.at[head], dsem),
                pltpu.async_copy(sx1.at[dblk], bx1.at[head], dsem),
                pltpu.async_copy(sy2.at[dblk], by2.at[head], dsem),
                pltpu.async_copy(sx2.at[dblk], bx2.at[head], dsem),
                pltpu.async_copy(sar.at[dblk], bar.at[head], dsem),
                pltpu.async_copy(skp.at[dblk], bkp.at[head], dsem),
            ]
            sdescs = [
                pltpu.async_copy(sy1.at[dsh], hy1.at[pl.ds(0, SHMAX)], hsem),
                pltpu.async_copy(sx1.at[dsh], hx1.at[pl.ds(0, SHMAX)], hsem),
                pltpu.async_copy(sy2.at[dsh], hy2.at[pl.ds(0, SHMAX)], hsem),
                pltpu.async_copy(sx2.at[dsh], hx2.at[pl.ds(0, SHMAX)], hsem),
                pltpu.async_copy(sar.at[dsh], har.at[pl.ds(0, SHMAX)], hsem),
                pltpu.async_copy(skp.at[dsh], hkp.at[pl.ds(0, SHMAX)], hsem),
            ]
            for d in bdescs:
                d.wait()

            lane16 = lax.broadcasted_iota(jnp.int32, (16,), 0)

            def intra_r(r, nk):
                kr = bkp[pl.ds(r, 16)][0]

                def kept(nk):
                    ry1 = by1[pl.ds(r, 16)][0]
                    rx1 = bx1[pl.ds(r, 16)][0]
                    ry2 = by2[pl.ds(r, 16)][0]
                    rx2 = bx2[pl.ds(r, 16)][0]
                    ra = bar[pl.ds(r, 16)][0]
                    my1[nk] = ry1
                    mx1[nk] = rx1
                    my2[nk] = ry2
                    mx2[nk] = rx2
                    mar[nk] = ra

                    def iv(v, _):
                        off = pl.ds(v * 16, 16)
                        supp = _iou_supp(ry1, rx1, ry2, rx2, ra,
                                         by1[off], bx1[off], by2[off],
                                         bx2[off], bar[off])
                        supp = supp & ((v * 16 + lane16) > r)
                        bkp[off] = jnp.where(supp, 0.0, bkp[off])
                        return 0

                    lax.fori_loop(r // 16, B // 16, iv, 0)
                    return nk + 1

                return lax.cond(kr > 0.0, kept, lambda nk: nk, nk)

            nk = lax.fori_loop(0, B, intra_r, 0)

            @pl.when(wid == lax.rem(blk, T))
            def _():
                pltpu.sync_copy(bkp.at[head], skp.at[dblk])

            nk4 = ((nk + 7) // 8) * 8

            def zpad(i, _):
                my1[i] = 0.0
                mx1[i] = 0.0
                my2[i] = 0.0
                mx2[i] = 0.0
                mar[i] = 0.0
                return 0

            lax.fori_loop(nk, nk4, zpad, 0)

            for d in sdescs:
                d.wait()

            def quad_body(k, _):
                rows = []
                for j in range(8):
                    i = 8 * k + j
                    rows.append((my1[i], mx1[i], my2[i], mx2[i], mar[i]))

                def sv_(v2, _):
                    for h in range(2):
                        off = pl.ds((v2 * 2 + h) * 16, 16)
                        y1v = hy1[off]; x1v = hx1[off]
                        y2v = hy2[off]; x2v = hx2[off]; av = har[off]
                        supp = None
                        for (ry1, rx1, ry2, rx2, ra) in rows:
                            s = _iou_supp(ry1, rx1, ry2, rx2, ra,
                                          y1v, x1v, y2v, x2v, av)
                            supp = s if supp is None else (supp | s)
                        hkp[off] = jnp.where(supp, 0.0, hkp[off])
                    return 0

                lax.fori_loop(0, (nv + 1) // 2, sv_, 0)
                return 0

            lax.fori_loop(0, nk4 // 8, quad_body, 0)

            for bit in (64, 32, 16, 8, 4, 2, 1):
                off_elems = (nv & ~(2 * bit - 1)) * 16

                @pl.when((nv & bit) != 0)
                def _(bit=bit, off_elems=off_elems):
                    pltpu.sync_copy(
                        hkp.at[pl.ds(pl.multiple_of(off_elems, 16), bit * 16)],
                        skp.at[pl.ds(pl.multiple_of(mystart + off_elems, 16),
                                     bit * 16)])

            plsc.subcore_barrier()
            return 0

        lax.fori_loop(0, NBLK, blk_body, 0)

        pltpu.sync_copy(sy1.at[dchunk], g1)
        pltpu.sync_copy(sx1.at[dchunk], g2)
        pltpu.sync_copy(sy2.at[dchunk], g3)
        pltpu.sync_copy(sx2.at[dchunk], g4)
        pltpu.sync_copy(ssv.at[dchunk], g5)
        pltpu.sync_copy(skp.at[dchunk], gk)

        def ccomp(v, _):
            off = pl.ds(v * 16, 16)
            m = gk[off] > 0.0
            g1[off] = jnp.where(m, g1[off], 0.0)
            g2[off] = jnp.where(m, g2[off], 0.0)
            g3[off] = jnp.where(m, g3[off], 0.0)
            g4[off] = jnp.where(m, g4[off], 0.0)
            g5[off] = jnp.where(m, g5[off], 0.0)
            return 0

        lax.fori_loop(0, VREGS, ccomp, 0)

        pltpu.sync_copy(g1, oy1.at[dchunk])
        pltpu.sync_copy(g2, ox1.at[dchunk])
        pltpu.sync_copy(g3, oy2.at[dchunk])
        pltpu.sync_copy(g4, ox2.at[dchunk])
        pltpu.sync_copy(g5, osv.at[dchunk])

    f32 = jnp.float32
    out_type = [jax.ShapeDtypeStruct((NP,), f32) for _ in range(5)]
    scratch = (
        [pltpu.VMEM_SHARED((SPAD,), f32) for _ in range(7)]
        + [pltpu.VMEM((BPAD,), f32) for _ in range(6)]
        + [pltpu.VMEM((SHMAX + 16,), f32) for _ in range(6)]
        + [pltpu.VMEM((CHUNK,), f32) for _ in range(7)]
        + [pltpu.VMEM((CHUNK // 128, 128), jnp.int32)]
        + [pltpu.SMEM((B + 8,), f32) for _ in range(5)]
        + [pltpu.SemaphoreType.DMA, pltpu.SemaphoreType.DMA]
    )
    return pl.kernel(body, out_type=out_type, mesh=mesh, scratch_types=scratch)


_nms_sc = _make_kernel()


@jax.jit
def kernel(boxes, scores):
    s = jnp.where(scores > SCORE_T, scores, -1.0)
    s_pad = jnp.concatenate([s, jnp.full((PAD,), -1.0, jnp.float32)])
    order = jnp.argsort(-s_pad).astype(jnp.int32)
    idxh = order.reshape(T, CHUNK // 128, 128)
    bp = jnp.concatenate([boxes, jnp.zeros((PAD, 4), jnp.float32)], axis=0)
    y1, x1, y2, x2, sv = _nms_sc(
        bp[:, 0], bp[:, 1], bp[:, 2], bp[:, 3], s_pad, idxh)
    return jnp.stack([y1, x1, y2, x2, sv], axis=1)[:N]
